# Initial kernel scaffold; baseline (speedup 1.0000x reference)
import jax
import jax.numpy as jnp
from jax.experimental import pallas as pl


def kernel(x, edge_index, edge_attr, target_index, target_batch_index, target_class, params):
    raise NotImplementedError("write your pallas kernel here")


# speedup vs baseline: 3.1924x; 3.1924x over previous
import functools

import jax
import jax.numpy as jnp
from jax import lax
from jax.experimental import pallas as pl
from jax.experimental.pallas import tpu as pltpu
from jax.experimental.pallas import tpu_sc as plsc

N = 10000
E = 320000
F_IN = 128
D = 16
B = 100
NOUT = 8

NW = 32
CH = 128
KE = 79
EP = NW * KE * CH
KT = 5
TP = NW * KT * CH
NP = N + 16
EB = 2048
NEB = EP // EB
NB = 2000


def _ln(x, g, b, eps=1e-5):
    mu = jnp.mean(x, axis=-1, keepdims=True)
    var = jnp.mean((x - mu) * (x - mu), axis=-1, keepdims=True)
    return (x - mu) * lax.rsqrt(var + eps) * g + b


def _full(shape):
    return pl.BlockSpec(shape, lambda i: (0, 0))


def _pre_body(x_ref, w1, b1, g1, be1, w2, b2, g2, be2, o_ref):
    h = jnp.dot(x_ref[...], w1[...], preferred_element_type=jnp.float32)
    h = jax.nn.relu(_ln(h + b1[...], g1[...], be1[...]))
    h = jnp.dot(h, w2[...], preferred_element_type=jnp.float32)
    o_ref[...] = jax.nn.relu(_ln(h + b2[...], g2[...], be2[...]))


def _pre_call(x, w1, b1, g1, be1, w2, b2, g2, be2):
    grid = N // NB
    return pl.pallas_call(
        _pre_body,
        grid=(grid,),
        in_specs=[
            pl.BlockSpec((NB, F_IN), lambda i: (i, 0)),
            _full((F_IN, D)), _full((1, D)), _full((1, D)), _full((1, D)),
            _full((D, D)), _full((1, D)), _full((1, D)), _full((1, D)),
        ],
        out_specs=pl.BlockSpec((NB, D), lambda i: (i, 0)),
        out_shape=jax.ShapeDtypeStruct((N, D), jnp.float32),
    )(x, w1, b1, g1, be1, w2, b2, g2, be2)


def _msg_body(ea_ref, xj_ref, w1, b1, g1, be1, w2, b2, g2, be2, R, S, o_ref):
    h = jnp.dot(ea_ref[...], w1[...], preferred_element_type=jnp.float32)
    h = jax.nn.relu(_ln(h + b1[...], g1[...], be1[...]))
    h2 = jnp.dot(h, w2[...], preferred_element_type=jnp.float32)
    h2 = _ln(h2 + b2[...], g2[...], be2[...])
    xjrep = jnp.dot(xj_ref[...], R[...], preferred_element_type=jnp.float32)
    o_ref[...] = jnp.dot(xjrep * h2, S[...], preferred_element_type=jnp.float32)


def _msg_call(ea, xj, w1, b1, g1, be1, w2, b2, g2, be2, R, S):
    return pl.pallas_call(
        _msg_body,
        grid=(NEB,),
        in_specs=[
            pl.BlockSpec((EB, 4), lambda i: (i, 0)),
            pl.BlockSpec((EB, D), lambda i: (i, 0)),
            _full((4, D)), _full((1, D)), _full((1, D)), _full((1, D)),
            _full((D, D * D)), _full((1, D * D)), _full((1, D * D)), _full((1, D * D)),
            _full((D, D * D)), _full((D * D, D)),
        ],
        out_specs=pl.BlockSpec((EB, D), lambda i: (i, 0)),
        out_shape=jax.ShapeDtypeStruct((EP, D), jnp.float32),
    )(ea, xj, w1, b1, g1, be1, w2, b2, g2, be2, R, S)


def _gru_body(a0, a1, d0, d1, h_ref, cb, wih, whh, bih, bhh, o_ref):
    deg = jnp.maximum(d0[:, :1] + d1[:, :1], 1.0)
    m = jax.nn.relu((a0[...] + a1[...]) / deg + cb[...])
    h = h_ref[...]
    gi = jnp.dot(m, wih[...], preferred_element_type=jnp.float32) + bih[...]
    gh = jnp.dot(h, whh[...], preferred_element_type=jnp.float32) + bhh[...]
    r = jax.nn.sigmoid(gi[:, :D] + gh[:, :D])
    z = jax.nn.sigmoid(gi[:, D:2 * D] + gh[:, D:2 * D])
    n = jnp.tanh(gi[:, 2 * D:] + r * gh[:, 2 * D:])
    o_ref[...] = (1.0 - z) * n + z * h


def _gru_call(a0, a1, d0, d1, h, cb, wih, whh, bih, bhh):
    grid = N // NB
    nspec = pl.BlockSpec((NB, D), lambda i: (i, 0))
    return pl.pallas_call(
        _gru_body,
        grid=(grid,),
        in_specs=[nspec, nspec, nspec, nspec, nspec,
                  _full((1, D)), _full((D, 3 * D)), _full((D, 3 * D)),
                  _full((1, 3 * D)), _full((1, 3 * D))],
        out_specs=nspec,
        out_shape=jax.ShapeDtypeStruct((N, D), jnp.float32),
    )(a0, a1, d0, d1, h, cb, wih, whh, bih, bhh)


def _s2s_body(out_ref, b_ref, wih, whh, bih, bhh, o_ref):
    out = out_ref[...]
    bidx = b_ref[...]
    lanes = lax.broadcasted_iota(jnp.int32, (1, 128), 1)
    onehot = (bidx == lanes).astype(jnp.float32)

    def seg(v):
        return lax.dot_general(onehot, v, (((0,), (0,)), ((), ())),
                               preferred_element_type=jnp.float32)

    q_star = jnp.zeros((128, 2 * D), jnp.float32)
    hx = jnp.zeros((128, D), jnp.float32)
    cx = jnp.zeros((128, D), jnp.float32)
    for _ in range(3):
        g_all = (jnp.dot(q_star, wih[...], preferred_element_type=jnp.float32)
                 + bih[...]
                 + jnp.dot(hx, whh[...], preferred_element_type=jnp.float32)
                 + bhh[...])
        i = jax.nn.sigmoid(g_all[:, :D])
        f = jax.nn.sigmoid(g_all[:, D:2 * D])
        gg = jnp.tanh(g_all[:, 2 * D:3 * D])
        o = jax.nn.sigmoid(g_all[:, 3 * D:])
        cx = f * cx + i * gg
        hx = o * jnp.tanh(cx)
        q = hx
        qb = jnp.dot(onehot, q, preferred_element_type=jnp.float32)
        e = jnp.sum(out * qb, axis=1, keepdims=True)
        emax = jnp.max(jnp.where(onehot > 0, e, -1e30), axis=0, keepdims=True)
        emax_b = jnp.sum(onehot * emax, axis=1, keepdims=True)
        ex = jnp.exp(e - emax_b)
        denom = seg(ex)
        denom_b = jnp.dot(onehot, denom, preferred_element_type=jnp.float32)
        a = ex / denom_b
        r_ = seg(a * out)
        q_star = jnp.concatenate([q, r_], axis=1)
    o_ref[...] = jnp.dot(onehot, q_star, preferred_element_type=jnp.float32)


def _s2s_call(out, batch2d, wih, whh, bih, bhh):
    return pl.pallas_call(
        _s2s_body,
        grid=(1,),
        in_specs=[
            pl.BlockSpec((N, D), lambda i: (0, 0)),
            pl.BlockSpec((N, 1), lambda i: (0, 0)),
            _full((2 * D, 4 * D)), _full((D, 4 * D)),
            _full((1, 4 * D)), _full((1, 4 * D)),
        ],
        out_specs=pl.BlockSpec((N, 2 * D), lambda i: (0, 0)),
        out_shape=jax.ShapeDtypeStruct((N, 2 * D), jnp.float32),
    )(out, batch2d, wih, whh, bih, bhh)


def _final_body(n0, n1, s2, tc_ref, w1, b1, g1, be1, w2, b2, g2, be2, w3, b3,
                o_ref):
    feat = jnp.concatenate([n0[...], n1[...], s2[...]], axis=1)
    p1 = jnp.dot(feat, w1[...], preferred_element_type=jnp.float32)
    p1 = jax.nn.relu(_ln(p1 + b1[...], g1[...], be1[...]))
    p2 = jnp.dot(p1, w2[...], preferred_element_type=jnp.float32)
    p2 = jax.nn.relu(_ln(p2 + b2[...], g2[...], be2[...]))
    pr = jnp.dot(p2, w3[...], preferred_element_type=jnp.float32) + b3[...]
    lanes = lax.broadcasted_iota(jnp.int32, (1, NOUT), 1)
    oh = (tc_ref[...] == lanes).astype(jnp.float32)
    o_ref[...] = jnp.sum(pr * oh, axis=1, keepdims=True)


def _final_call(n0, n1, s2, tc2d, w1, b1, g1, be1, w2, b2, g2, be2, w3, b3):
    grid = N // NB
    F4 = 4 * D
    return pl.pallas_call(
        _final_body,
        grid=(grid,),
        in_specs=[
            pl.BlockSpec((NB, D), lambda i: (i, 0)),
            pl.BlockSpec((NB, D), lambda i: (i, 0)),
            pl.BlockSpec((NB, 2 * D), lambda i: (i, 0)),
            pl.BlockSpec((NB, 1), lambda i: (i, 0)),
            _full((F4, F4)), _full((1, F4)), _full((1, F4)), _full((1, F4)),
            _full((F4, F4)), _full((1, F4)), _full((1, F4)), _full((1, F4)),
            _full((F4, NOUT)), _full((1, NOUT)),
        ],
        out_specs=pl.BlockSpec((NB, 1), lambda i: (i, 0)),
        out_shape=jax.ShapeDtypeStruct((N, 1), jnp.float32),
    )(n0, n1, s2, tc2d, w1, b1, g1, be1, w2, b2, g2, be2, w3, b3)


@functools.lru_cache(maxsize=None)
def _make_sc_gather(k):
    @functools.partial(
        pl.kernel,
        out_type=jax.ShapeDtypeStruct((NW, k, CH, D), jnp.float32),
        mesh=plsc.VectorSubcoreMesh(core_axis_name="c", subcore_axis_name="s"),
        compiler_params=pltpu.CompilerParams(use_tc_tiling_on_sc=False),
        scratch_types=[
            pltpu.VMEM((k, CH), jnp.int32),
            pltpu.VMEM((CH, D), jnp.float32),
            pltpu.SemaphoreType.DMA,
        ],
    )
    def sc_gather(table_hbm, idx_hbm, out_hbm, idx_v, rows_v, sem):
        wid = lax.axis_index("s") * 2 + lax.axis_index("c")
        pltpu.sync_copy(idx_hbm.at[wid], idx_v)

        def body(j, carry):
            pltpu.async_copy(table_hbm.at[idx_v.at[j]], rows_v, sem).wait()
            pltpu.sync_copy(rows_v, out_hbm.at[wid, j])
            return carry

        lax.fori_loop(0, k, body, 0)

    return sc_gather


@functools.lru_cache(maxsize=None)
def _make_sc_scatter():
    @functools.partial(
        pl.kernel,
        out_type=jax.ShapeDtypeStruct((2, NP, D), jnp.float32),
        mesh=plsc.VectorSubcoreMesh(core_axis_name="c", subcore_axis_name="s"),
        compiler_params=pltpu.CompilerParams(use_tc_tiling_on_sc=False),
        scratch_types=[
            pltpu.VMEM((KE, CH), jnp.int32),
            pltpu.VMEM((CH, D), jnp.float32),
            pltpu.VMEM_SHARED((NP, D), jnp.float32),
        ],
    )
    def sc_scatter(msg_hbm, dst_hbm, zeros_hbm, out_hbm, idx_v, rows_v, acc_sh):
        cid = lax.axis_index("c")
        sid = lax.axis_index("s")
        wid = sid * 2 + cid

        @pl.when(sid == 0)
        def _():
            pltpu.sync_copy(zeros_hbm, acc_sh)

        plsc.subcore_barrier()
        pltpu.sync_copy(dst_hbm.at[wid], idx_v)

        def body(j, carry):
            pltpu.sync_copy(msg_hbm.at[wid, j], rows_v)
            pltpu.sync_copy(rows_v, acc_sh.at[idx_v.at[j]], add=True)
            return carry

        lax.fori_loop(0, KE, body, 0)
        plsc.subcore_barrier()
        rp = NP // 16
        pltpu.sync_copy(acc_sh.at[pl.ds(sid * rp, rp)],
                        out_hbm.at[cid, pl.ds(sid * rp, rp)])

    return sc_scatter


@functools.lru_cache(maxsize=None)
def _make_sc_degree():
    @functools.partial(
        pl.kernel,
        out_type=jax.ShapeDtypeStruct((2, NP, D), jnp.float32),
        mesh=plsc.VectorSubcoreMesh(core_axis_name="c", subcore_axis_name="s"),
        compiler_params=pltpu.CompilerParams(use_tc_tiling_on_sc=False),
        scratch_types=[
            pltpu.VMEM((KE, CH), jnp.int32),
            pltpu.VMEM((CH, D), jnp.float32),
            pltpu.VMEM_SHARED((NP, D), jnp.float32),
        ],
    )
    def sc_degree(dst_hbm, ones_hbm, zeros_hbm, out_hbm, idx_v, rows_v, acc_sh):
        cid = lax.axis_index("c")
        sid = lax.axis_index("s")
        wid = sid * 2 + cid

        @pl.when(sid == 0)
        def _():
            pltpu.sync_copy(zeros_hbm, acc_sh)

        plsc.subcore_barrier()
        pltpu.sync_copy(dst_hbm.at[wid], idx_v)
        pltpu.sync_copy(ones_hbm, rows_v)

        def body(j, carry):
            pltpu.sync_copy(rows_v, acc_sh.at[idx_v.at[j]], add=True)
            return carry

        lax.fori_loop(0, KE, body, 0)
        plsc.subcore_barrier()
        rp = NP // 16
        pltpu.sync_copy(acc_sh.at[pl.ds(sid * rp, rp)],
                        out_hbm.at[cid, pl.ds(sid * rp, rp)])

    return sc_degree


def kernel(x, edge_index, edge_attr, target_index, target_batch_index,
           target_class, params):
    p = params
    f32 = jnp.float32

    def row(v):
        return v.reshape(1, -1).astype(f32)

    src = edge_index[0].astype(jnp.int32)
    dst = edge_index[1].astype(jnp.int32)
    pad_e = EP - E
    src_p = jnp.concatenate([src, jnp.zeros((pad_e,), jnp.int32)]
                            ).reshape(NW, KE, CH)
    dst_p = jnp.concatenate([dst, jnp.full((pad_e,), N, jnp.int32)]
                            ).reshape(NW, KE, CH)
    ea_p = jnp.concatenate(
        [edge_attr.astype(f32), jnp.zeros((pad_e, 4), f32)])
    tgt = jnp.concatenate(
        [target_index[0].astype(jnp.int32), target_index[1].astype(jnp.int32),
         jnp.zeros((TP - 2 * N,), jnp.int32)]).reshape(NW, KT, CH)
    batch2d = target_batch_index.astype(jnp.int32).reshape(N, 1)
    tc2d = target_class.astype(jnp.int32).reshape(N, 1)

    zeros_np = jnp.zeros((NP, D), f32)
    ones_ch = jnp.ones((CH, D), f32)

    R = jnp.kron(jnp.eye(D, dtype=f32), jnp.ones((1, D), f32))
    S = jnp.kron(jnp.ones((D, 1), f32), jnp.eye(D, dtype=f32))

    pre_args = (p['pre_w1'].T.astype(f32), row(p['pre_b1']), row(p['pre_g1']),
                row(p['pre_be1']), p['pre_w2'].T.astype(f32), row(p['pre_b2']),
                row(p['pre_g2']), row(p['pre_be2']))
    enc_args = (p['enc_w1'].T.astype(f32), row(p['enc_b1']), row(p['enc_g1']),
                row(p['enc_be1']), p['enc_w2'].T.astype(f32), row(p['enc_b2']),
                row(p['enc_g2']), row(p['enc_be2']))
    gru_args = (row(p['conv_b']), p['gru_wih'].T.astype(f32),
                p['gru_whh'].T.astype(f32), row(p['gru_bih']),
                row(p['gru_bhh']))
    lstm_args = (p['lstm_wih'].T.astype(f32), p['lstm_whh'].T.astype(f32),
                 row(p['lstm_bih']), row(p['lstm_bhh']))
    pr_args = (p['pr_w1'].T.astype(f32), row(p['pr_b1']), row(p['pr_g1']),
               row(p['pr_be1']), p['pr_w2'].T.astype(f32), row(p['pr_b2']),
               row(p['pr_g2']), row(p['pr_be2']), p['pr_w3'].T.astype(f32),
               row(p['pr_b3']))

    gather_e = _make_sc_gather(KE)
    gather_t = _make_sc_gather(KT)

    h = _pre_call(x.astype(f32), *pre_args)

    deg = _make_sc_degree()(dst_p, ones_ch, zeros_np)
    d0 = deg[0, :N]
    d1 = deg[1, :N]

    for _ in range(3):
        h_pad = jnp.concatenate([h, jnp.zeros((NP - N, D), f32)])
        xj = gather_e(h_pad, src_p).reshape(EP, D)
        msg = _msg_call(ea_p, xj, *enc_args, R, S).reshape(NW, KE, CH, D)
        acc = _make_sc_scatter()(msg, dst_p, zeros_np)
        h = _gru_call(acc[0, :N], acc[1, :N], d0, d1, h, *gru_args)

    h_pad = jnp.concatenate([h, jnp.zeros((NP - N, D), f32)])
    rows = gather_t(h_pad, tgt).reshape(TP, D)
    node0 = rows[:N]
    node1 = rows[N:2 * N]
    s2s0 = _s2s_call(h, batch2d, *lstm_args)
    outv = _final_call(node0, node1, s2s0, tc2d, *pr_args)
    return outv[:, 0]


# speedup vs baseline: 3.2919x; 1.0312x over previous
import functools

import jax
import jax.numpy as jnp
from jax import lax
from jax.experimental import pallas as pl
from jax.experimental.pallas import tpu as pltpu
from jax.experimental.pallas import tpu_sc as plsc

N = 10000
E = 320000
F_IN = 128
D = 16
B = 100
NOUT = 8

NW = 32
CH = 128
KE = 80
EP = NW * KE * CH
KT = 8
TP = NW * KT * CH
G = 8
NP = N + 16
EB = 2048
NEB = EP // EB
NB = 2000


def _ln(x, g, b, eps=1e-5):
    mu = jnp.mean(x, axis=-1, keepdims=True)
    var = jnp.mean((x - mu) * (x - mu), axis=-1, keepdims=True)
    return (x - mu) * lax.rsqrt(var + eps) * g + b


def _full(shape):
    return pl.BlockSpec(shape, lambda i: (0, 0))


def _pre_body(x_ref, w1, b1, g1, be1, w2, b2, g2, be2, o_ref):
    h = jnp.dot(x_ref[...], w1[...], preferred_element_type=jnp.float32)
    h = jax.nn.relu(_ln(h + b1[...], g1[...], be1[...]))
    h = jnp.dot(h, w2[...], preferred_element_type=jnp.float32)
    o_ref[...] = jax.nn.relu(_ln(h + b2[...], g2[...], be2[...]))


def _pre_call(x, w1, b1, g1, be1, w2, b2, g2, be2):
    grid = N // NB
    return pl.pallas_call(
        _pre_body,
        grid=(grid,),
        in_specs=[
            pl.BlockSpec((NB, F_IN), lambda i: (i, 0)),
            _full((F_IN, D)), _full((1, D)), _full((1, D)), _full((1, D)),
            _full((D, D)), _full((1, D)), _full((1, D)), _full((1, D)),
        ],
        out_specs=pl.BlockSpec((NB, D), lambda i: (i, 0)),
        out_shape=jax.ShapeDtypeStruct((N, D), jnp.float32),
    )(x, w1, b1, g1, be1, w2, b2, g2, be2)


def _msg_body(ea_ref, xj_ref, w1, b1, g1, be1, w2, b2, g2, be2, R, S, o_ref):
    h = jnp.dot(ea_ref[...], w1[...], preferred_element_type=jnp.float32)
    h = jax.nn.relu(_ln(h + b1[...], g1[...], be1[...]))
    h2 = jnp.dot(h, w2[...], preferred_element_type=jnp.float32)
    h2 = _ln(h2 + b2[...], g2[...], be2[...])
    xjrep = jnp.dot(xj_ref[...], R[...], preferred_element_type=jnp.float32)
    o_ref[...] = jnp.dot(xjrep * h2, S[...], preferred_element_type=jnp.float32)


def _msg_call(ea, xj, w1, b1, g1, be1, w2, b2, g2, be2, R, S):
    return pl.pallas_call(
        _msg_body,
        grid=(NEB,),
        in_specs=[
            pl.BlockSpec((EB, 4), lambda i: (i, 0)),
            pl.BlockSpec((EB, D), lambda i: (i, 0)),
            _full((4, D)), _full((1, D)), _full((1, D)), _full((1, D)),
            _full((D, D * D)), _full((1, D * D)), _full((1, D * D)), _full((1, D * D)),
            _full((D, D * D)), _full((D * D, D)),
        ],
        out_specs=pl.BlockSpec((EB, D), lambda i: (i, 0)),
        out_shape=jax.ShapeDtypeStruct((EP, D), jnp.float32),
    )(ea, xj, w1, b1, g1, be1, w2, b2, g2, be2, R, S)


def _gru_body(a0, a1, d0, d1, h_ref, cb, wih, whh, bih, bhh, o_ref):
    deg = jnp.maximum(d0[:, :1] + d1[:, :1], 1.0)
    m = jax.nn.relu((a0[...] + a1[...]) / deg + cb[...])
    h = h_ref[...]
    gi = jnp.dot(m, wih[...], preferred_element_type=jnp.float32) + bih[...]
    gh = jnp.dot(h, whh[...], preferred_element_type=jnp.float32) + bhh[...]
    r = jax.nn.sigmoid(gi[:, :D] + gh[:, :D])
    z = jax.nn.sigmoid(gi[:, D:2 * D] + gh[:, D:2 * D])
    n = jnp.tanh(gi[:, 2 * D:] + r * gh[:, 2 * D:])
    o_ref[...] = (1.0 - z) * n + z * h


def _gru_call(a0, a1, d0, d1, h, cb, wih, whh, bih, bhh):
    grid = N // NB
    nspec = pl.BlockSpec((NB, D), lambda i: (i, 0))
    return pl.pallas_call(
        _gru_body,
        grid=(grid,),
        in_specs=[nspec, nspec, nspec, nspec, nspec,
                  _full((1, D)), _full((D, 3 * D)), _full((D, 3 * D)),
                  _full((1, 3 * D)), _full((1, 3 * D))],
        out_specs=nspec,
        out_shape=jax.ShapeDtypeStruct((N, D), jnp.float32),
    )(a0, a1, d0, d1, h, cb, wih, whh, bih, bhh)


def _s2s_body(out_ref, b_ref, n0_ref, n1_ref, tc_ref, wih, whh, bih, bhh,
              w1, b1, g1, be1, w2, b2, g2, be2, w3, b3, o_ref):
    out = out_ref[...]
    bidx = b_ref[...]
    lanes = lax.broadcasted_iota(jnp.int32, (1, 128), 1)
    onehot = (bidx == lanes).astype(jnp.float32)

    def seg(v):
        return lax.dot_general(onehot, v, (((0,), (0,)), ((), ())),
                               preferred_element_type=jnp.float32)

    q_star = jnp.zeros((128, 2 * D), jnp.float32)
    hx = jnp.zeros((128, D), jnp.float32)
    cx = jnp.zeros((128, D), jnp.float32)
    for _ in range(3):
        g_all = (jnp.dot(q_star, wih[...], preferred_element_type=jnp.float32)
                 + bih[...]
                 + jnp.dot(hx, whh[...], preferred_element_type=jnp.float32)
                 + bhh[...])
        i = jax.nn.sigmoid(g_all[:, :D])
        f = jax.nn.sigmoid(g_all[:, D:2 * D])
        gg = jnp.tanh(g_all[:, 2 * D:3 * D])
        o = jax.nn.sigmoid(g_all[:, 3 * D:])
        cx = f * cx + i * gg
        hx = o * jnp.tanh(cx)
        q = hx
        qb = jnp.dot(onehot, q, preferred_element_type=jnp.float32)
        e = jnp.sum(out * qb, axis=1, keepdims=True)
        emax = jnp.max(jnp.where(onehot > 0, e, -1e30), axis=0, keepdims=True)
        emax_b = jnp.sum(onehot * emax, axis=1, keepdims=True)
        ex = jnp.exp(e - emax_b)
        denom = seg(ex)
        denom_b = jnp.dot(onehot, denom, preferred_element_type=jnp.float32)
        a = ex / denom_b
        r_ = seg(a * out)
        q_star = jnp.concatenate([q, r_], axis=1)
    s2 = jnp.dot(onehot, q_star, preferred_element_type=jnp.float32)

    feat = jnp.concatenate([n0_ref[...], n1_ref[...], s2], axis=1)
    p1 = jnp.dot(feat, w1[...], preferred_element_type=jnp.float32)
    p1 = jax.nn.relu(_ln(p1 + b1[...], g1[...], be1[...]))
    p2 = jnp.dot(p1, w2[...], preferred_element_type=jnp.float32)
    p2 = jax.nn.relu(_ln(p2 + b2[...], g2[...], be2[...]))
    pr = jnp.dot(p2, w3[...], preferred_element_type=jnp.float32) + b3[...]
    lanes8 = lax.broadcasted_iota(jnp.int32, (1, NOUT), 1)
    oh = (tc_ref[...] == lanes8).astype(jnp.float32)
    o_ref[...] = jnp.sum(pr * oh, axis=1, keepdims=True)


def _s2s_final_call(out, batch2d, n0, n1, tc2d, wih, whh, bih, bhh,
                    w1, b1, g1, be1, w2, b2, g2, be2, w3, b3):
    F4 = 4 * D
    return pl.pallas_call(
        _s2s_body,
        grid=(1,),
        in_specs=[
            pl.BlockSpec((N, D), lambda i: (0, 0)),
            pl.BlockSpec((N, 1), lambda i: (0, 0)),
            pl.BlockSpec((N, D), lambda i: (0, 0)),
            pl.BlockSpec((N, D), lambda i: (0, 0)),
            pl.BlockSpec((N, 1), lambda i: (0, 0)),
            _full((2 * D, 4 * D)), _full((D, 4 * D)),
            _full((1, 4 * D)), _full((1, 4 * D)),
            _full((F4, F4)), _full((1, F4)), _full((1, F4)), _full((1, F4)),
            _full((F4, F4)), _full((1, F4)), _full((1, F4)), _full((1, F4)),
            _full((F4, NOUT)), _full((1, NOUT)),
        ],
        out_specs=pl.BlockSpec((N, 1), lambda i: (0, 0)),
        out_shape=jax.ShapeDtypeStruct((N, 1), jnp.float32),
    )(out, batch2d, n0, n1, tc2d, wih, whh, bih, bhh,
      w1, b1, g1, be1, w2, b2, g2, be2, w3, b3)


@functools.lru_cache(maxsize=None)
def _make_sc_gather(k):
    ng = k // G

    @functools.partial(
        pl.kernel,
        out_type=jax.ShapeDtypeStruct((NW, k, CH, D), jnp.float32),
        mesh=plsc.VectorSubcoreMesh(core_axis_name="c", subcore_axis_name="s"),
        compiler_params=pltpu.CompilerParams(use_tc_tiling_on_sc=False),
        scratch_types=[
            pltpu.VMEM((k, CH), jnp.int32),
            pltpu.VMEM((2, G, CH, D), jnp.float32),
            pltpu.SemaphoreType.DMA,
            pltpu.SemaphoreType.DMA((2,)),
        ],
    )
    def sc_gather(table_hbm, idx_hbm, out_hbm, idx_v, rows_v, gsem, wbsem):
        wid = lax.axis_index("s") * 2 + lax.axis_index("c")
        pltpu.sync_copy(idx_hbm.at[wid], idx_v)

        def body(t, carry):
            b = lax.rem(t, 2)

            @pl.when(t >= 2)
            def _():
                pltpu.make_async_copy(
                    rows_v.at[b],
                    out_hbm.at[wid, pl.ds((t - 2) * G, G)],
                    wbsem.at[b]).wait()

            descs = [
                pltpu.async_copy(table_hbm.at[idx_v.at[t * G + c]],
                                 rows_v.at[b, c], gsem)
                for c in range(G)
            ]
            for d in descs:
                d.wait()
            pltpu.async_copy(rows_v.at[b], out_hbm.at[wid, pl.ds(t * G, G)],
                             wbsem.at[b])
            return carry

        lax.fori_loop(0, ng, body, 0)
        for g in range(max(ng - 2, 0), ng):
            pltpu.make_async_copy(
                rows_v.at[g % 2],
                out_hbm.at[wid, pl.ds(g * G, G)],
                wbsem.at[g % 2]).wait()

    return sc_gather


@functools.lru_cache(maxsize=None)
def _make_sc_scatter():
    @functools.partial(
        pl.kernel,
        out_type=jax.ShapeDtypeStruct((2, NP, D), jnp.float32),
        mesh=plsc.VectorSubcoreMesh(core_axis_name="c", subcore_axis_name="s"),
        compiler_params=pltpu.CompilerParams(use_tc_tiling_on_sc=False),
        scratch_types=[
            pltpu.VMEM((KE, CH), jnp.int32),
            pltpu.VMEM((2, G, CH, D), jnp.float32),
            pltpu.VMEM_SHARED((NP, D), jnp.float32),
            pltpu.SemaphoreType.DMA((2,)),
            pltpu.SemaphoreType.DMA,
        ],
    )
    def sc_scatter(msg_hbm, dst_hbm, zeros_hbm, out_hbm, idx_v, rows_v,
                   acc_sh, lsem, ssem):
        cid = lax.axis_index("c")
        sid = lax.axis_index("s")
        wid = sid * 2 + cid
        ng = KE // G

        @pl.when(sid == 0)
        def _():
            pltpu.sync_copy(zeros_hbm, acc_sh)

        plsc.subcore_barrier()
        pltpu.sync_copy(dst_hbm.at[wid], idx_v)
        pltpu.async_copy(msg_hbm.at[wid, pl.ds(0, G)], rows_v.at[0],
                         lsem.at[0])

        def body(t, carry):
            b = lax.rem(t, 2)
            pltpu.make_async_copy(msg_hbm.at[wid, pl.ds(t * G, G)],
                                  rows_v.at[b], lsem.at[b]).wait()

            @pl.when(t + 1 < ng)
            def _():
                pltpu.async_copy(msg_hbm.at[wid, pl.ds((t + 1) * G, G)],
                                 rows_v.at[1 - b], lsem.at[1 - b])

            descs = [
                pltpu.async_copy(rows_v.at[b, c],
                                 acc_sh.at[idx_v.at[t * G + c]], ssem,
                                 add=True)
                for c in range(G)
            ]
            for d in descs:
                d.wait()
            return carry

        lax.fori_loop(0, ng, body, 0)
        plsc.subcore_barrier()
        rp = NP // 16
        pltpu.sync_copy(acc_sh.at[pl.ds(sid * rp, rp)],
                        out_hbm.at[cid, pl.ds(sid * rp, rp)])

    return sc_scatter


@functools.lru_cache(maxsize=None)
def _make_sc_degree():
    @functools.partial(
        pl.kernel,
        out_type=jax.ShapeDtypeStruct((2, NP, D), jnp.float32),
        mesh=plsc.VectorSubcoreMesh(core_axis_name="c", subcore_axis_name="s"),
        compiler_params=pltpu.CompilerParams(use_tc_tiling_on_sc=False),
        scratch_types=[
            pltpu.VMEM((KE, CH), jnp.int32),
            pltpu.VMEM((CH, D), jnp.float32),
            pltpu.VMEM_SHARED((NP, D), jnp.float32),
            pltpu.SemaphoreType.DMA,
        ],
    )
    def sc_degree(dst_hbm, ones_hbm, zeros_hbm, out_hbm, idx_v, rows_v,
                  acc_sh, ssem):
        cid = lax.axis_index("c")
        sid = lax.axis_index("s")
        wid = sid * 2 + cid

        @pl.when(sid == 0)
        def _():
            pltpu.sync_copy(zeros_hbm, acc_sh)

        plsc.subcore_barrier()
        pltpu.sync_copy(dst_hbm.at[wid], idx_v)
        pltpu.sync_copy(ones_hbm, rows_v)

        def body(t, carry):
            descs = [
                pltpu.async_copy(rows_v, acc_sh.at[idx_v.at[t * G + c]],
                                 ssem, add=True)
                for c in range(G)
            ]
            for d in descs:
                d.wait()
            return carry

        lax.fori_loop(0, KE // G, body, 0)
        plsc.subcore_barrier()
        rp = NP // 16
        pltpu.sync_copy(acc_sh.at[pl.ds(sid * rp, rp)],
                        out_hbm.at[cid, pl.ds(sid * rp, rp)])

    return sc_degree


def kernel(x, edge_index, edge_attr, target_index, target_batch_index,
           target_class, params):
    p = params
    f32 = jnp.float32

    def row(v):
        return v.reshape(1, -1).astype(f32)

    src = edge_index[0].astype(jnp.int32)
    dst = edge_index[1].astype(jnp.int32)
    pad_e = EP - E
    src_p = jnp.concatenate([src, jnp.zeros((pad_e,), jnp.int32)]
                            ).reshape(NW, KE, CH)
    dst_p = jnp.concatenate([dst, jnp.full((pad_e,), N, jnp.int32)]
                            ).reshape(NW, KE, CH)
    ea_p = jnp.concatenate(
        [edge_attr.astype(f32), jnp.zeros((pad_e, 4), f32)])
    tgt = jnp.concatenate(
        [target_index[0].astype(jnp.int32), target_index[1].astype(jnp.int32),
         jnp.zeros((TP - 2 * N,), jnp.int32)]).reshape(NW, KT, CH)
    batch2d = target_batch_index.astype(jnp.int32).reshape(N, 1)
    tc2d = target_class.astype(jnp.int32).reshape(N, 1)

    zeros_np = jnp.zeros((NP, D), f32)
    ones_ch = jnp.ones((CH, D), f32)

    R = jnp.kron(jnp.eye(D, dtype=f32), jnp.ones((1, D), f32))
    S = jnp.kron(jnp.ones((D, 1), f32), jnp.eye(D, dtype=f32))

    pre_args = (p['pre_w1'].T.astype(f32), row(p['pre_b1']), row(p['pre_g1']),
                row(p['pre_be1']), p['pre_w2'].T.astype(f32), row(p['pre_b2']),
                row(p['pre_g2']), row(p['pre_be2']))
    enc_args = (p['enc_w1'].T.astype(f32), row(p['enc_b1']), row(p['enc_g1']),
                row(p['enc_be1']), p['enc_w2'].T.astype(f32), row(p['enc_b2']),
                row(p['enc_g2']), row(p['enc_be2']))
    gru_args = (row(p['conv_b']), p['gru_wih'].T.astype(f32),
                p['gru_whh'].T.astype(f32), row(p['gru_bih']),
                row(p['gru_bhh']))
    lstm_args = (p['lstm_wih'].T.astype(f32), p['lstm_whh'].T.astype(f32),
                 row(p['lstm_bih']), row(p['lstm_bhh']))
    pr_args = (p['pr_w1'].T.astype(f32), row(p['pr_b1']), row(p['pr_g1']),
               row(p['pr_be1']), p['pr_w2'].T.astype(f32), row(p['pr_b2']),
               row(p['pr_g2']), row(p['pr_be2']), p['pr_w3'].T.astype(f32),
               row(p['pr_b3']))

    gather_e = _make_sc_gather(KE)
    gather_t = _make_sc_gather(KT)

    h = _pre_call(x.astype(f32), *pre_args)

    deg = _make_sc_degree()(dst_p, ones_ch, zeros_np)
    d0 = deg[0, :N]
    d1 = deg[1, :N]

    for _ in range(3):
        h_pad = jnp.concatenate([h, jnp.zeros((NP - N, D), f32)])
        xj = gather_e(h_pad, src_p).reshape(EP, D)
        msg = _msg_call(ea_p, xj, *enc_args, R, S).reshape(NW, KE, CH, D)
        acc = _make_sc_scatter()(msg, dst_p, zeros_np)
        h = _gru_call(acc[0, :N], acc[1, :N], d0, d1, h, *gru_args)

    h_pad = jnp.concatenate([h, jnp.zeros((NP - N, D), f32)])
    rows = gather_t(h_pad, tgt).reshape(TP, D)
    node0 = rows[:N]
    node1 = rows[N:2 * N]
    outv = _s2s_final_call(h, batch2d, node0, node1, tc2d, *lstm_args,
                           *pr_args)
    return outv[:, 0]
